# Initial kernel scaffold; baseline (speedup 1.0000x reference)
import functools

import jax
import jax.numpy as jnp
from jax import lax
from jax.experimental import pallas as pl
from jax.experimental.pallas import tpu as pltpu
from jax.experimental.pallas import tpu_sc as plsc

B = 8
HW = 384 * 384
NTPI = 4
CHUNK = HW // NTPI
NPIECE = 4
PIECE = CHUNK // NPIECE
VECS_PER_PIECE = PIECE // 16
SENT = jnp.int32(0x7FFFFFFF)

SHIFT = (22, 14, 7, 0)
BMASK = (0x1FF, 0xFF, 0x7F, 0x7F)
NBINS = (512, 256, 128, 128)
PCHK = (31, 22, 14, 7)
PWID = (9, 8, 7, 7)
MAXB = 512

_mesh = plsc.VectorSubcoreMesh(core_axis_name="c", subcore_axis_name="s")


@functools.partial(
    pl.kernel,
    out_type=jax.ShapeDtypeStruct((B, 16), jnp.float32),
    mesh=_mesh,
    scratch_types=[
        pltpu.VMEM((PIECE,), jnp.float32),
        pltpu.VMEM((PIECE,), jnp.float32),
        pltpu.VMEM((PIECE,), jnp.int32),
        pltpu.VMEM((CHUNK,), jnp.int32),
        pltpu.VMEM((MAXB * 16,), jnp.int32),
        pltpu.VMEM((MAXB * 16,), jnp.float32),
        pltpu.VMEM((MAXB,), jnp.int32),
        pltpu.VMEM((MAXB,), jnp.float32),
        pltpu.VMEM((NTPI, MAXB), jnp.int32),
        pltpu.VMEM((NTPI, MAXB), jnp.float32),
        pltpu.VMEM((16,), jnp.float32),
        pltpu.VMEM_SHARED((16, MAXB), jnp.int32),
        pltpu.VMEM_SHARED((16, MAXB), jnp.float32),
    ],
)
def _tmae_sc(pred_hbm, targ_hbm, mask_hbm, out_hbm,
             pv, tv, mv, errb, cnt_ls, sum_ls, cnt_c, sum_c,
             cnt_cb, sum_cb, outv, shared_cnt, shared_sum):
    c = lax.axis_index("c")
    s = lax.axis_index("s")
    il = s // NTPI
    tg = s % NTPI
    img = c * (16 // NTPI) + il
    base = img * HW + tg * CHUNK

    iota16 = lax.iota(jnp.int32, 16)
    ones_i = jnp.ones((16,), jnp.int32)
    zeros_i = jnp.zeros((16,), jnp.int32)
    zeros_f = jnp.zeros((16,), jnp.float32)

    def zero_hist(nb):
        def zbody(i, _):
            cnt_ls[pl.ds(i * 16, 16)] = zeros_i
            sum_ls[pl.ds(i * 16, 16)] = zeros_f
            return 0
        lax.fori_loop(0, nb, zbody, 0)

    def hist_update(ebits, valid, vals, level):
        bins = jnp.right_shift(ebits, SHIFT[level]) & jnp.int32(BMASK[level])
        idx = bins * jnp.int32(16) + iota16
        plsc.addupdate_scatter(cnt_ls, [idx], ones_i, mask=valid)
        plsc.addupdate_scatter(sum_ls, [idx], vals, mask=valid)

    def publish_and_combine(nb):
        def cbody(b, _):
            cnt_c[b] = jnp.sum(cnt_ls[pl.ds(b * 16, 16)])
            sum_c[b] = jnp.sum(sum_ls[pl.ds(b * 16, 16)])
            return 0
        lax.fori_loop(0, nb, cbody, 0)
        pltpu.sync_copy(cnt_c, shared_cnt.at[s])
        pltpu.sync_copy(sum_c, shared_sum.at[s])
        plsc.subcore_barrier()
        pltpu.sync_copy(shared_cnt.at[pl.ds(il * NTPI, NTPI)], cnt_cb)
        pltpu.sync_copy(shared_sum.at[pl.ds(il * NTPI, NTPI)], sum_cb)
        plsc.subcore_barrier()

    def group_vec(i):
        cvec = (cnt_cb[0, pl.ds(i * 16, 16)] + cnt_cb[1, pl.ds(i * 16, 16)]
                + cnt_cb[2, pl.ds(i * 16, 16)] + cnt_cb[3, pl.ds(i * 16, 16)])
        svec = (sum_cb[0, pl.ds(i * 16, 16)] + sum_cb[1, pl.ds(i * 16, 16)]
                + sum_cb[2, pl.ds(i * 16, 16)] + sum_cb[3, pl.ds(i * 16, 16)])
        return cvec, svec

    def find_bin(nb, kk):
        def fbody(i, carry):
            cum, cums, found, binf, cbel, sbel = carry
            cvec, svec = group_vec(i)
            inc = plsc.cumsum(cvec)
            hit = (inc + cum) >= kk
            first = jnp.min(jnp.where(hit, iota16, jnp.int32(16)))
            newhit = jnp.logical_and(first < 16, found == 0)
            below = iota16 < first
            exc_c = jnp.sum(jnp.where(below, cvec, jnp.int32(0)))
            exc_s = jnp.sum(jnp.where(below, svec, jnp.float32(0.0)))
            binf = jnp.where(newhit, i * 16 + first, binf)
            cbel = jnp.where(newhit, cum + exc_c, cbel)
            sbel = jnp.where(newhit, cums + exc_s, sbel)
            found = jnp.where(newhit, jnp.int32(1), found)
            cum = cum + jnp.sum(cvec)
            cums = cums + jnp.sum(svec)
            return cum, cums, found, binf, cbel, sbel
        init = (jnp.int32(0), jnp.float32(0.0), jnp.int32(0),
                jnp.int32(0), jnp.int32(0), jnp.float32(0.0))
        out = lax.fori_loop(0, nb // 16, fbody, init)
        return out[3], out[4], out[5]

    zero_hist(NBINS[0])

    def piece_body(j, _):
        off = base + j * PIECE
        pltpu.sync_copy(pred_hbm.at[pl.ds(off, PIECE)], pv)
        pltpu.sync_copy(targ_hbm.at[pl.ds(off, PIECE)], tv)
        pltpu.sync_copy(mask_hbm.at[pl.ds(off, PIECE)], mv)

        def vbody(i, _):
            p = pv[pl.ds(i * 16, 16)]
            t = tv[pl.ds(i * 16, 16)]
            m = mv[pl.ds(i * 16, 16)]
            d = jnp.abs(p - t)
            db = plsc.bitcast(d, jnp.int32)
            valid = m > 0
            eb = jnp.where(valid, db, SENT)
            errb[pl.ds((j * VECS_PER_PIECE + i) * 16, 16)] = eb
            hist_update(eb, valid, d, 0)
            return 0
        lax.fori_loop(0, VECS_PER_PIECE, vbody, 0)
        return 0

    lax.fori_loop(0, NPIECE, piece_body, 0)
    publish_and_combine(NBINS[0])

    def mbody(i, acc):
        cvec, _ = group_vec(i)
        return acc + jnp.sum(cvec)
    M = lax.fori_loop(0, NBINS[0] // 16, mbody, jnp.int32(0))
    Mf = M.astype(jnp.float32)
    k = (Mf * jnp.float32(0.8)).astype(jnp.int32)

    b0, c0, s0 = find_bin(NBINS[0], k)
    kk = k - c0
    prefix = b0
    sum_below = s0

    for level in (1, 2, 3):
        zero_hist(NBINS[level])

        def lbody(i, _, _level=level, _prefix=prefix):
            eb = errb[pl.ds(i * 16, 16)]
            valid = jnp.logical_and(
                eb != SENT,
                jnp.right_shift(eb, PCHK[_level]) == _prefix)
            vals = plsc.bitcast(eb, jnp.float32)
            hist_update(eb, valid, vals, _level)
            return 0
        lax.fori_loop(0, CHUNK // 16, lbody, 0)
        publish_and_combine(NBINS[level])
        bl, cl, sl = find_bin(NBINS[level], kk)
        kk = kk - cl
        sum_below = sum_below + sl
        prefix = prefix * jnp.int32(1 << PWID[level]) + bl

    t = jnp.max(plsc.bitcast(jnp.full((16,), prefix, jnp.int32), jnp.float32))
    image_sum = sum_below + kk.astype(jnp.float32) * t

    @pl.when(tg == 0)
    def _():
        outv[...] = jnp.where(iota16 == 0, image_sum,
                              jnp.where(iota16 == 1, Mf, jnp.float32(0.0)))
        pltpu.sync_copy(outv, out_hbm.at[img])


def kernel(prediction, target, mask):
    pf = prediction.reshape(-1)
    tf = target.reshape(-1)
    mf = mask.reshape(-1)
    out = _tmae_sc(pf, tf, mf)
    image_sum = out[:, 0]
    M = out[:, 1]
    divisor = jnp.sum(M * jnp.float32(0.8))
    return jnp.where(divisor == 0,
                     jnp.float32(0.0),
                     jnp.sum(image_sum) / jnp.maximum(divisor, jnp.float32(1e-12)))


# speedup vs baseline: 8.9123x; 8.9123x over previous
import functools

import jax
import jax.numpy as jnp
from jax import lax
from jax.experimental import pallas as pl
from jax.experimental.pallas import tpu as pltpu
from jax.experimental.pallas import tpu_sc as plsc

B = 8
HW = 384 * 384
NTPI = 4
CHUNK = HW // NTPI
NPIECE = 4
PIECE = CHUNK // NPIECE
VECS_PER_PIECE = PIECE // 16
SENT = 0x7FFFFFFF

SHIFT = (22, 14, 7, 0)
BMASK = (0x1FF, 0xFF, 0x7F, 0x7F)
NBINS = (512, 256, 128, 128)
PCHK = (31, 22, 14, 7)
PWID = (9, 8, 7, 7)
MAXB = 512

_mesh = plsc.VectorSubcoreMesh(core_axis_name="c", subcore_axis_name="s")


@functools.partial(
    pl.kernel,
    out_type=jax.ShapeDtypeStruct((B, 16), jnp.float32),
    mesh=_mesh,
    compiler_params=pltpu.CompilerParams(needs_layout_passes=False),
    scratch_types=[
        pltpu.VMEM((PIECE,), jnp.float32),
        pltpu.VMEM((PIECE,), jnp.float32),
        pltpu.VMEM((PIECE,), jnp.int32),
        pltpu.VMEM((CHUNK,), jnp.int32),
        pltpu.VMEM((MAXB * 16,), jnp.int32),
        pltpu.VMEM((MAXB * 16,), jnp.float32),
        pltpu.VMEM((MAXB,), jnp.int32),
        pltpu.VMEM((MAXB,), jnp.float32),
        pltpu.VMEM((NTPI, MAXB), jnp.int32),
        pltpu.VMEM((NTPI, MAXB), jnp.float32),
        pltpu.VMEM((16,), jnp.float32),
        pltpu.VMEM_SHARED((16, MAXB), jnp.int32),
        pltpu.VMEM_SHARED((16, MAXB), jnp.float32),
    ],
)
def _tmae_sc(pred_hbm, targ_hbm, mask_hbm, out_hbm,
             pv, tv, mv, errb, cnt_ls, sum_ls, cnt_c, sum_c,
             cnt_cb, sum_cb, outv, shared_cnt, shared_sum):
    c = lax.axis_index("c")
    s = lax.axis_index("s")
    il = s // NTPI
    tg = s % NTPI
    img = c * (16 // NTPI) + il
    base = img * HW + tg * CHUNK

    iota16 = lax.iota(jnp.int32, 16)
    ones_i = jnp.ones((16,), jnp.int32)
    zeros_i = jnp.zeros((16,), jnp.int32)
    zeros_f = jnp.zeros((16,), jnp.float32)

    def zero_hist(nb):
        def zbody(i, _):
            cnt_ls[pl.ds(i * 16, 16)] = zeros_i
            sum_ls[pl.ds(i * 16, 16)] = zeros_f
            return 0
        lax.fori_loop(0, nb, zbody, 0)

    def hist_update(ebits, valid, vals, level):
        bins = jnp.right_shift(ebits, SHIFT[level]) & jnp.int32(BMASK[level])
        idx = bins * jnp.int32(16) + iota16
        plsc.addupdate_scatter(cnt_ls, [idx], ones_i, mask=valid)
        plsc.addupdate_scatter(sum_ls, [idx], vals, mask=valid)

    def publish_and_combine(nb):
        def cbody(i, _):
            acc_c = zeros_i
            acc_s = zeros_f
            for j in range(16):
                cc = jnp.sum(cnt_ls[pl.ds((i * 16 + j) * 16, 16)])
                ss = jnp.sum(sum_ls[pl.ds((i * 16 + j) * 16, 16)])
                acc_c = jnp.where(iota16 == j, cc, acc_c)
                acc_s = jnp.where(iota16 == j, ss, acc_s)
            cnt_c[pl.ds(i * 16, 16)] = acc_c
            sum_c[pl.ds(i * 16, 16)] = acc_s
            return 0
        lax.fori_loop(0, nb // 16, cbody, 0)
        pltpu.sync_copy(cnt_c, shared_cnt.at[s])
        pltpu.sync_copy(sum_c, shared_sum.at[s])
        plsc.subcore_barrier()
        pltpu.sync_copy(shared_cnt.at[pl.ds(il * NTPI, NTPI)], cnt_cb)
        pltpu.sync_copy(shared_sum.at[pl.ds(il * NTPI, NTPI)], sum_cb)
        plsc.subcore_barrier()

    def group_vec(i):
        cvec = (cnt_cb[0, pl.ds(i * 16, 16)] + cnt_cb[1, pl.ds(i * 16, 16)]
                + cnt_cb[2, pl.ds(i * 16, 16)] + cnt_cb[3, pl.ds(i * 16, 16)])
        svec = (sum_cb[0, pl.ds(i * 16, 16)] + sum_cb[1, pl.ds(i * 16, 16)]
                + sum_cb[2, pl.ds(i * 16, 16)] + sum_cb[3, pl.ds(i * 16, 16)])
        return cvec, svec

    def find_bin(nb, kk):
        def fbody(i, carry):
            cum, cums, found, binf, cbel, sbel = carry
            cvec, svec = group_vec(i)
            inc = plsc.cumsum(cvec)
            hit = (inc + cum) >= kk
            first = jnp.min(jnp.where(hit, iota16, jnp.int32(16)))
            newhit = jnp.logical_and(first < 16, found == 0)
            below = iota16 < first
            exc_c = jnp.sum(jnp.where(below, cvec, jnp.int32(0)))
            exc_s = jnp.sum(jnp.where(below, svec, jnp.float32(0.0)))
            binf = jnp.where(newhit, i * 16 + first, binf)
            cbel = jnp.where(newhit, cum + exc_c, cbel)
            sbel = jnp.where(newhit, cums + exc_s, sbel)
            found = jnp.where(newhit, jnp.int32(1), found)
            cum = cum + jnp.sum(cvec)
            cums = cums + jnp.sum(svec)
            return cum, cums, found, binf, cbel, sbel
        init = (jnp.int32(0), jnp.float32(0.0), jnp.int32(0),
                jnp.int32(0), jnp.int32(0), jnp.float32(0.0))
        out = lax.fori_loop(0, nb // 16, fbody, init)
        return out[3], out[4], out[5]

    zero_hist(NBINS[0])

    def piece_body(j, _):
        off = base + j * PIECE
        pltpu.sync_copy(pred_hbm.at[pl.ds(off, PIECE)], pv)
        pltpu.sync_copy(targ_hbm.at[pl.ds(off, PIECE)], tv)
        pltpu.sync_copy(mask_hbm.at[pl.ds(off, PIECE)], mv)

        def vbody(i, _):
            p = pv[pl.ds(i * 16, 16)]
            t = tv[pl.ds(i * 16, 16)]
            m = mv[pl.ds(i * 16, 16)]
            d = jnp.abs(p - t)
            db = lax.bitcast_convert_type(d, jnp.int32)
            valid = m > 0
            eb = jnp.where(valid, db, jnp.int32(SENT))
            errb[pl.ds((j * VECS_PER_PIECE + i) * 16, 16)] = eb
            hist_update(eb, valid, d, 0)
            return 0
        lax.fori_loop(0, VECS_PER_PIECE, vbody, 0)
        return 0

    lax.fori_loop(0, NPIECE, piece_body, 0)
    publish_and_combine(NBINS[0])

    def mbody(i, acc):
        cvec, _ = group_vec(i)
        return acc + jnp.sum(cvec)
    M = lax.fori_loop(0, NBINS[0] // 16, mbody, jnp.int32(0))
    Mf = M.astype(jnp.float32)
    k = (Mf * jnp.float32(0.8)).astype(jnp.int32)

    b0, c0, s0 = find_bin(NBINS[0], k)
    kk = k - c0
    prefix = b0
    sum_below = s0

    for level in (1, 2, 3):
        zero_hist(NBINS[level])

        def lbody(i, _, _level=level, _prefix=prefix):
            eb = errb[pl.ds(i * 16, 16)]
            valid = jnp.logical_and(
                eb != jnp.int32(SENT),
                jnp.right_shift(eb, PCHK[_level]) == _prefix)
            vals = lax.bitcast_convert_type(eb, jnp.float32)
            hist_update(eb, valid, vals, _level)
            return 0
        lax.fori_loop(0, CHUNK // 16, lbody, 0)
        publish_and_combine(NBINS[level])
        bl, cl, sl = find_bin(NBINS[level], kk)
        kk = kk - cl
        sum_below = sum_below + sl
        prefix = prefix * jnp.int32(1 << PWID[level]) + bl

    t = jnp.max(lax.bitcast_convert_type(jnp.full((16,), prefix, jnp.int32), jnp.float32))
    image_sum = sum_below + kk.astype(jnp.float32) * t

    @pl.when(tg == 0)
    def _():
        outv[...] = jnp.where(iota16 == 0, image_sum,
                              jnp.where(iota16 == 1, Mf, jnp.float32(0.0)))
        pltpu.sync_copy(outv, out_hbm.at[img])


def kernel(prediction, target, mask):
    pf = prediction.reshape(-1)
    tf = target.reshape(-1)
    mf = mask.reshape(-1)
    out = _tmae_sc(pf, tf, mf)
    image_sum = out[:, 0]
    M = out[:, 1]
    divisor = jnp.sum(M * jnp.float32(0.8))
    return jnp.where(divisor == 0,
                     jnp.float32(0.0),
                     jnp.sum(image_sum) / jnp.maximum(divisor, jnp.float32(1e-12)))


# speedup vs baseline: 12.0555x; 1.3527x over previous
import functools

import jax
import jax.numpy as jnp
from jax import lax
from jax.experimental import pallas as pl
from jax.experimental.pallas import tpu as pltpu
from jax.experimental.pallas import tpu_sc as plsc

B = 8
HW = 384 * 384
NTPI = 4
CHUNK = HW // NTPI
NPIECE = 4
PIECE = CHUNK // NPIECE
VECS_PER_PIECE = PIECE // 16
SENT = 0x7FFFFFFF

SHIFT = (22, 14, 7, 0)
BMASK = (0x1FF, 0xFF, 0x7F, 0x7F)
NBINS = (512, 256, 128, 128)
PCHK = (31, 22, 14, 7)
PWID = (9, 8, 7, 7)
MAXB = 512

_mesh = plsc.VectorSubcoreMesh(core_axis_name="c", subcore_axis_name="s")


@functools.partial(
    pl.kernel,
    out_type=jax.ShapeDtypeStruct((B, 16), jnp.float32),
    mesh=_mesh,
    compiler_params=pltpu.CompilerParams(needs_layout_passes=False),
    scratch_types=[
        pltpu.VMEM((PIECE,), jnp.float32),
        pltpu.VMEM((PIECE,), jnp.float32),
        pltpu.VMEM((PIECE,), jnp.int32),
        pltpu.VMEM((CHUNK,), jnp.int32),
        pltpu.VMEM((MAXB * 16,), jnp.int32),
        pltpu.VMEM((MAXB * 16,), jnp.float32),
        pltpu.VMEM((MAXB,), jnp.int32),
        pltpu.VMEM((MAXB,), jnp.float32),
        pltpu.VMEM((NTPI, MAXB), jnp.int32),
        pltpu.VMEM((NTPI, MAXB), jnp.float32),
        pltpu.VMEM((16,), jnp.float32),
        pltpu.VMEM((CHUNK + 16,), jnp.int32),
        pltpu.VMEM_SHARED((16, MAXB), jnp.int32),
        pltpu.VMEM_SHARED((16, MAXB), jnp.float32),
    ],
)
def _tmae_sc(pred_hbm, targ_hbm, mask_hbm, out_hbm,
             pv, tv, mv, errb, cnt_ls, sum_ls, cnt_c, sum_c,
             cnt_cb, sum_cb, outv, cand, shared_cnt, shared_sum):
    c = lax.axis_index("c")
    s = lax.axis_index("s")
    il = s // NTPI
    tg = s % NTPI
    img = c * (16 // NTPI) + il
    base = img * HW + tg * CHUNK

    iota16 = lax.iota(jnp.int32, 16)
    ones_i = jnp.ones((16,), jnp.int32)
    zeros_i = jnp.zeros((16,), jnp.int32)
    zeros_f = jnp.zeros((16,), jnp.float32)

    def zero_hist(nb):
        def zbody(i, _):
            cnt_ls[pl.ds(i * 16, 16)] = zeros_i
            sum_ls[pl.ds(i * 16, 16)] = zeros_f
            return 0
        lax.fori_loop(0, nb, zbody, 0)

    def hist_update(ebits, valid, vals, level):
        bins = jnp.right_shift(ebits, SHIFT[level]) & jnp.int32(BMASK[level])
        idx = bins * jnp.int32(16) + iota16
        plsc.addupdate_scatter(cnt_ls, [idx], ones_i, mask=valid)
        plsc.addupdate_scatter(sum_ls, [idx], vals, mask=valid)

    def publish_and_combine(nb):
        def cbody(i, _):
            acc_c = zeros_i
            acc_s = zeros_f
            for j in range(16):
                cc = jnp.sum(cnt_ls[pl.ds((i * 16 + j) * 16, 16)])
                ss = jnp.sum(sum_ls[pl.ds((i * 16 + j) * 16, 16)])
                acc_c = jnp.where(iota16 == j, cc, acc_c)
                acc_s = jnp.where(iota16 == j, ss, acc_s)
            cnt_c[pl.ds(i * 16, 16)] = acc_c
            sum_c[pl.ds(i * 16, 16)] = acc_s
            return 0
        lax.fori_loop(0, nb // 16, cbody, 0)
        pltpu.sync_copy(cnt_c, shared_cnt.at[s])
        pltpu.sync_copy(sum_c, shared_sum.at[s])
        plsc.subcore_barrier()
        pltpu.sync_copy(shared_cnt.at[pl.ds(il * NTPI, NTPI)], cnt_cb)
        pltpu.sync_copy(shared_sum.at[pl.ds(il * NTPI, NTPI)], sum_cb)
        plsc.subcore_barrier()

    def group_vec(i):
        cvec = (cnt_cb[0, pl.ds(i * 16, 16)] + cnt_cb[1, pl.ds(i * 16, 16)]
                + cnt_cb[2, pl.ds(i * 16, 16)] + cnt_cb[3, pl.ds(i * 16, 16)])
        svec = (sum_cb[0, pl.ds(i * 16, 16)] + sum_cb[1, pl.ds(i * 16, 16)]
                + sum_cb[2, pl.ds(i * 16, 16)] + sum_cb[3, pl.ds(i * 16, 16)])
        return cvec, svec

    def find_bin(nb, kk):
        def fbody(i, carry):
            cum, cums, found, binf, cbel, sbel = carry
            cvec, svec = group_vec(i)
            inc = plsc.cumsum(cvec)
            hit = (inc + cum) >= kk
            first = jnp.min(jnp.where(hit, iota16, jnp.int32(16)))
            newhit = jnp.logical_and(first < 16, found == 0)
            below = iota16 < first
            exc_c = jnp.sum(jnp.where(below, cvec, jnp.int32(0)))
            exc_s = jnp.sum(jnp.where(below, svec, jnp.float32(0.0)))
            binf = jnp.where(newhit, i * 16 + first, binf)
            cbel = jnp.where(newhit, cum + exc_c, cbel)
            sbel = jnp.where(newhit, cums + exc_s, sbel)
            found = jnp.where(newhit, jnp.int32(1), found)
            cum = cum + jnp.sum(cvec)
            cums = cums + jnp.sum(svec)
            return cum, cums, found, binf, cbel, sbel
        init = (jnp.int32(0), jnp.float32(0.0), jnp.int32(0),
                jnp.int32(0), jnp.int32(0), jnp.float32(0.0))
        out = lax.fori_loop(0, nb // 16, fbody, init)
        return out[3], out[4], out[5]

    zero_hist(NBINS[0])

    def piece_body(j, _):
        off = base + j * PIECE
        pltpu.sync_copy(pred_hbm.at[pl.ds(off, PIECE)], pv)
        pltpu.sync_copy(targ_hbm.at[pl.ds(off, PIECE)], tv)
        pltpu.sync_copy(mask_hbm.at[pl.ds(off, PIECE)], mv)

        def vbody(i, _):
            p = pv[pl.ds(i * 16, 16)]
            t = tv[pl.ds(i * 16, 16)]
            m = mv[pl.ds(i * 16, 16)]
            d = jnp.abs(p - t)
            db = lax.bitcast_convert_type(d, jnp.int32)
            valid = m > 0
            eb = jnp.where(valid, db, jnp.int32(SENT))
            errb[pl.ds((j * VECS_PER_PIECE + i) * 16, 16)] = eb
            hist_update(eb, valid, d, 0)
            return 0
        lax.fori_loop(0, VECS_PER_PIECE, vbody, 0)
        return 0

    lax.fori_loop(0, NPIECE, piece_body, 0)
    publish_and_combine(NBINS[0])

    def mbody(i, acc):
        cvec, _ = group_vec(i)
        return acc + jnp.sum(cvec)
    M = lax.fori_loop(0, NBINS[0] // 16, mbody, jnp.int32(0))
    Mf = M.astype(jnp.float32)
    k = (Mf * jnp.float32(0.8)).astype(jnp.int32)

    b0, c0, s0 = find_bin(NBINS[0], k)
    kk = k - c0
    prefix = b0
    sum_below = s0

    zero_hist(NBINS[1])

    def l1body(i, off):
        eb = errb[pl.ds(i * 16, 16)]
        match = jnp.logical_and(
            eb != jnp.int32(SENT),
            jnp.right_shift(eb, PCHK[1]) == prefix)
        vals = lax.bitcast_convert_type(eb, jnp.float32)
        hist_update(eb, match, vals, 1)
        plsc.store_compressed(cand.at[pl.ds(off, 16)], eb, mask=match)
        return off + jnp.max(plsc.all_reduce_population_count(match))
    nc = lax.fori_loop(0, CHUNK // 16, l1body, jnp.int32(0))
    publish_and_combine(NBINS[1])
    b1, c1, s1 = find_bin(NBINS[1], kk)
    kk = kk - c1
    sum_below = sum_below + s1
    prefix = prefix * jnp.int32(1 << PWID[1]) + b1

    nvec = (nc + jnp.int32(15)) // jnp.int32(16)
    for level in (2, 3):
        zero_hist(NBINS[level])

        def lbody(i, _, _level=level, _prefix=prefix):
            eb = cand[pl.ds(i * 16, 16)]
            inb = (i * 16 + iota16) < nc
            valid = jnp.logical_and(
                inb, jnp.right_shift(eb, PCHK[_level]) == _prefix)
            vals = lax.bitcast_convert_type(eb, jnp.float32)
            hist_update(eb, valid, vals, _level)
            return 0
        lax.fori_loop(0, nvec, lbody, 0)
        publish_and_combine(NBINS[level])
        bl, cl, sl = find_bin(NBINS[level], kk)
        kk = kk - cl
        sum_below = sum_below + sl
        prefix = prefix * jnp.int32(1 << PWID[level]) + bl

    t = jnp.max(lax.bitcast_convert_type(jnp.full((16,), prefix, jnp.int32), jnp.float32))
    image_sum = sum_below + kk.astype(jnp.float32) * t

    @pl.when(tg == 0)
    def _():
        outv[...] = jnp.where(iota16 == 0, image_sum,
                              jnp.where(iota16 == 1, Mf, jnp.float32(0.0)))
        pltpu.sync_copy(outv, out_hbm.at[img])


def kernel(prediction, target, mask):
    pf = prediction.reshape(-1)
    tf = target.reshape(-1)
    mf = mask.reshape(-1)
    out = _tmae_sc(pf, tf, mf)
    image_sum = out[:, 0]
    M = out[:, 1]
    divisor = jnp.sum(M * jnp.float32(0.8))
    return jnp.where(divisor == 0,
                     jnp.float32(0.0),
                     jnp.sum(image_sum) / jnp.maximum(divisor, jnp.float32(1e-12)))


# speedup vs baseline: 12.2341x; 1.0148x over previous
import functools

import jax
import jax.numpy as jnp
from jax import lax
from jax.experimental import pallas as pl
from jax.experimental.pallas import tpu as pltpu
from jax.experimental.pallas import tpu_sc as plsc

B = 8
HW = 384 * 384
NTPI = 4
CHUNK = HW // NTPI
NPIECE = 4
PIECE = CHUNK // NPIECE
VECS_PER_PIECE = PIECE // 16
SENT = 0x7FFFFFFF

SHIFT = (22, 14, 7, 0)
BMASK = (0x1FF, 0xFF, 0x7F, 0x7F)
NBINS = (512, 256, 128, 128)
PCHK = (31, 22, 14, 7)
PWID = (9, 8, 7, 7)
MAXB = 512

_mesh = plsc.VectorSubcoreMesh(core_axis_name="c", subcore_axis_name="s")


@functools.partial(
    pl.kernel,
    out_type=jax.ShapeDtypeStruct((B, 16), jnp.float32),
    mesh=_mesh,
    compiler_params=pltpu.CompilerParams(needs_layout_passes=False),
    scratch_types=[
        pltpu.VMEM((PIECE,), jnp.float32),
        pltpu.VMEM((PIECE,), jnp.float32),
        pltpu.VMEM((PIECE,), jnp.int32),
        pltpu.VMEM((CHUNK,), jnp.int32),
        pltpu.VMEM((MAXB * 16,), jnp.int32),
        pltpu.VMEM((MAXB * 16,), jnp.float32),
        pltpu.VMEM((MAXB,), jnp.int32),
        pltpu.VMEM((MAXB,), jnp.float32),
        pltpu.VMEM((NTPI, MAXB), jnp.int32),
        pltpu.VMEM((NTPI, MAXB), jnp.float32),
        pltpu.VMEM((16,), jnp.float32),
        pltpu.VMEM((CHUNK + 16,), jnp.int32),
        pltpu.VMEM_SHARED((16, MAXB), jnp.int32),
        pltpu.VMEM_SHARED((16, MAXB), jnp.float32),
    ],
)
def _tmae_sc(pred_hbm, targ_hbm, mask_hbm, out_hbm,
             pv, tv, mv, errb, cnt_ls, sum_ls, cnt_c, sum_c,
             cnt_cb, sum_cb, outv, cand, shared_cnt, shared_sum):
    c = lax.axis_index("c")
    s = lax.axis_index("s")
    il = s // NTPI
    tg = s % NTPI
    img = c * (16 // NTPI) + il
    base = img * HW + tg * CHUNK

    iota16 = lax.iota(jnp.int32, 16)
    ones_i = jnp.ones((16,), jnp.int32)
    zeros_i = jnp.zeros((16,), jnp.int32)
    zeros_f = jnp.zeros((16,), jnp.float32)

    def zero_hist(nb):
        def zbody(i, _):
            cnt_ls[pl.ds(i * 16, 16)] = zeros_i
            sum_ls[pl.ds(i * 16, 16)] = zeros_f
            return 0
        lax.fori_loop(0, nb, zbody, 0, unroll=4)

    def hist_update(ebits, valid, vals, level):
        bins = jnp.right_shift(ebits, SHIFT[level]) & jnp.int32(BMASK[level])
        idx = bins * jnp.int32(16) + iota16
        plsc.addupdate_scatter(cnt_ls, [idx], ones_i, mask=valid)
        plsc.addupdate_scatter(sum_ls, [idx], vals, mask=valid)

    def publish_and_combine(nb):
        def cbody(i, _):
            acc_c = zeros_i
            acc_s = zeros_f
            for j in range(16):
                cc = jnp.sum(cnt_ls[pl.ds((i * 16 + j) * 16, 16)])
                ss = jnp.sum(sum_ls[pl.ds((i * 16 + j) * 16, 16)])
                acc_c = jnp.where(iota16 == j, cc, acc_c)
                acc_s = jnp.where(iota16 == j, ss, acc_s)
            cnt_c[pl.ds(i * 16, 16)] = acc_c
            sum_c[pl.ds(i * 16, 16)] = acc_s
            return 0
        lax.fori_loop(0, nb // 16, cbody, 0)
        pltpu.sync_copy(cnt_c, shared_cnt.at[s])
        pltpu.sync_copy(sum_c, shared_sum.at[s])
        plsc.subcore_barrier()
        pltpu.sync_copy(shared_cnt.at[pl.ds(il * NTPI, NTPI)], cnt_cb)
        pltpu.sync_copy(shared_sum.at[pl.ds(il * NTPI, NTPI)], sum_cb)
        plsc.subcore_barrier()

    def group_vec(i):
        cvec = (cnt_cb[0, pl.ds(i * 16, 16)] + cnt_cb[1, pl.ds(i * 16, 16)]
                + cnt_cb[2, pl.ds(i * 16, 16)] + cnt_cb[3, pl.ds(i * 16, 16)])
        svec = (sum_cb[0, pl.ds(i * 16, 16)] + sum_cb[1, pl.ds(i * 16, 16)]
                + sum_cb[2, pl.ds(i * 16, 16)] + sum_cb[3, pl.ds(i * 16, 16)])
        return cvec, svec

    def find_bin(nb, kk):
        def fbody(i, carry):
            cum, cums, found, binf, cbel, sbel = carry
            cvec, svec = group_vec(i)
            inc = plsc.cumsum(cvec)
            hit = (inc + cum) >= kk
            first = jnp.min(jnp.where(hit, iota16, jnp.int32(16)))
            newhit = jnp.logical_and(first < 16, found == 0)
            below = iota16 < first
            exc_c = jnp.sum(jnp.where(below, cvec, jnp.int32(0)))
            exc_s = jnp.sum(jnp.where(below, svec, jnp.float32(0.0)))
            binf = jnp.where(newhit, i * 16 + first, binf)
            cbel = jnp.where(newhit, cum + exc_c, cbel)
            sbel = jnp.where(newhit, cums + exc_s, sbel)
            found = jnp.where(newhit, jnp.int32(1), found)
            cum = cum + jnp.sum(cvec)
            cums = cums + jnp.sum(svec)
            return cum, cums, found, binf, cbel, sbel
        init = (jnp.int32(0), jnp.float32(0.0), jnp.int32(0),
                jnp.int32(0), jnp.int32(0), jnp.float32(0.0))
        out = lax.fori_loop(0, nb // 16, fbody, init)
        return out[3], out[4], out[5]

    zero_hist(NBINS[0])

    def piece_body(j, _):
        off = base + j * PIECE
        pltpu.sync_copy(pred_hbm.at[pl.ds(off, PIECE)], pv)
        pltpu.sync_copy(targ_hbm.at[pl.ds(off, PIECE)], tv)
        pltpu.sync_copy(mask_hbm.at[pl.ds(off, PIECE)], mv)

        def vbody(i, _):
            p = pv[pl.ds(i * 16, 16)]
            t = tv[pl.ds(i * 16, 16)]
            m = mv[pl.ds(i * 16, 16)]
            d = jnp.abs(p - t)
            db = lax.bitcast_convert_type(d, jnp.int32)
            valid = m > 0
            eb = jnp.where(valid, db, jnp.int32(SENT))
            errb[pl.ds((j * VECS_PER_PIECE + i) * 16, 16)] = eb
            hist_update(eb, valid, d, 0)
            return 0
        lax.fori_loop(0, VECS_PER_PIECE, vbody, 0, unroll=8)
        return 0

    lax.fori_loop(0, NPIECE, piece_body, 0)
    publish_and_combine(NBINS[0])

    def mbody(i, acc):
        cvec, _ = group_vec(i)
        return acc + jnp.sum(cvec)
    M = lax.fori_loop(0, NBINS[0] // 16, mbody, jnp.int32(0))
    Mf = M.astype(jnp.float32)
    k = (Mf * jnp.float32(0.8)).astype(jnp.int32)

    b0, c0, s0 = find_bin(NBINS[0], k)
    kk = k - c0
    prefix = b0
    sum_below = s0

    zero_hist(NBINS[1])

    def l1body(i, off):
        eb = errb[pl.ds(i * 16, 16)]
        match = jnp.logical_and(
            eb != jnp.int32(SENT),
            jnp.right_shift(eb, PCHK[1]) == prefix)
        vals = lax.bitcast_convert_type(eb, jnp.float32)
        hist_update(eb, match, vals, 1)
        plsc.store_compressed(cand.at[pl.ds(off, 16)], eb, mask=match)
        return off + jnp.max(plsc.all_reduce_population_count(match))
    nc = lax.fori_loop(0, CHUNK // 16, l1body, jnp.int32(0))
    publish_and_combine(NBINS[1])
    b1, c1, s1 = find_bin(NBINS[1], kk)
    kk = kk - c1
    sum_below = sum_below + s1
    prefix = prefix * jnp.int32(1 << PWID[1]) + b1

    nvec = (nc + jnp.int32(15)) // jnp.int32(16)
    for level in (2, 3):
        zero_hist(NBINS[level])

        def lbody(i, _, _level=level, _prefix=prefix):
            eb = cand[pl.ds(i * 16, 16)]
            inb = (i * 16 + iota16) < nc
            valid = jnp.logical_and(
                inb, jnp.right_shift(eb, PCHK[_level]) == _prefix)
            vals = lax.bitcast_convert_type(eb, jnp.float32)
            hist_update(eb, valid, vals, _level)
            return 0
        lax.fori_loop(0, nvec, lbody, 0)
        publish_and_combine(NBINS[level])
        bl, cl, sl = find_bin(NBINS[level], kk)
        kk = kk - cl
        sum_below = sum_below + sl
        prefix = prefix * jnp.int32(1 << PWID[level]) + bl

    t = jnp.max(lax.bitcast_convert_type(jnp.full((16,), prefix, jnp.int32), jnp.float32))
    image_sum = sum_below + kk.astype(jnp.float32) * t

    @pl.when(tg == 0)
    def _():
        outv[...] = jnp.where(iota16 == 0, image_sum,
                              jnp.where(iota16 == 1, Mf, jnp.float32(0.0)))
        pltpu.sync_copy(outv, out_hbm.at[img])


def kernel(prediction, target, mask):
    pf = prediction.reshape(-1)
    tf = target.reshape(-1)
    mf = mask.reshape(-1)
    out = _tmae_sc(pf, tf, mf)
    image_sum = out[:, 0]
    M = out[:, 1]
    divisor = jnp.sum(M * jnp.float32(0.8))
    return jnp.where(divisor == 0,
                     jnp.float32(0.0),
                     jnp.sum(image_sum) / jnp.maximum(divisor, jnp.float32(1e-12)))


# speedup vs baseline: 12.4505x; 1.0177x over previous
import functools

import jax
import jax.numpy as jnp
from jax import lax
from jax.experimental import pallas as pl
from jax.experimental.pallas import tpu as pltpu
from jax.experimental.pallas import tpu_sc as plsc

B = 8
HW = 384 * 384
NTPI = 4
CHUNK = HW // NTPI
NPIECE = 4
PIECE = CHUNK // NPIECE
VECS_PER_PIECE = PIECE // 16
SENT = 0x7FFFFFFF

SHIFT = (22, 14, 7, 0)
BMASK = (0x1FF, 0xFF, 0x7F, 0x7F)
NBINS = (512, 256, 128, 128)
PCHK = (31, 22, 14, 7)
PWID = (9, 8, 7, 7)
MAXB = 512

_mesh = plsc.VectorSubcoreMesh(core_axis_name="c", subcore_axis_name="s")


@functools.partial(
    pl.kernel,
    out_type=jax.ShapeDtypeStruct((B, 16), jnp.float32),
    mesh=_mesh,
    compiler_params=pltpu.CompilerParams(needs_layout_passes=False),
    scratch_types=[
        pltpu.VMEM((PIECE,), jnp.float32),
        pltpu.VMEM((PIECE,), jnp.float32),
        pltpu.VMEM((PIECE,), jnp.int32),
        pltpu.VMEM((CHUNK,), jnp.int32),
        pltpu.VMEM((MAXB * 16,), jnp.int32),
        pltpu.VMEM((MAXB,), jnp.int32),
        pltpu.VMEM((NTPI, MAXB), jnp.int32),
        pltpu.VMEM((CHUNK + 16,), jnp.int32),
        pltpu.VMEM((16,), jnp.float32),
        pltpu.VMEM((NTPI, 16), jnp.float32),
        pltpu.VMEM((16,), jnp.float32),
        pltpu.VMEM_SHARED((16, MAXB), jnp.int32),
        pltpu.VMEM_SHARED((16, 16), jnp.float32),
    ],
)
def _tmae_sc(pred_hbm, targ_hbm, mask_hbm, out_hbm,
             pv, tv, mv, errb, cnt_ls, cnt_c, cnt_cb, cand,
             psum_pub, psum_cb, outv, shared_cnt, shared_psum):
    c = lax.axis_index("c")
    s = lax.axis_index("s")
    il = s // NTPI
    tg = s % NTPI
    img = c * (16 // NTPI) + il
    base = img * HW + tg * CHUNK

    iota16 = lax.iota(jnp.int32, 16)
    ones_i = jnp.ones((16,), jnp.int32)
    zeros_i = jnp.zeros((16,), jnp.int32)
    zeros_f = jnp.zeros((16,), jnp.float32)

    def zero_hist(nb):
        def zbody(i, _):
            cnt_ls[pl.ds(i * 16, 16)] = zeros_i
            return 0
        lax.fori_loop(0, nb, zbody, 0, unroll=4)

    def hist_count(ebits, valid, level):
        bins = jnp.right_shift(ebits, SHIFT[level]) & jnp.int32(BMASK[level])
        idx = bins * jnp.int32(16) + iota16
        plsc.addupdate_scatter(cnt_ls, [idx], ones_i, mask=valid)

    def publish_and_combine(nb):
        def cbody(i, _):
            acc_c = zeros_i
            for j in range(16):
                cc = jnp.sum(cnt_ls[pl.ds((i * 16 + j) * 16, 16)])
                acc_c = jnp.where(iota16 == j, cc, acc_c)
            cnt_c[pl.ds(i * 16, 16)] = acc_c
            return 0
        lax.fori_loop(0, nb // 16, cbody, 0)
        pltpu.sync_copy(cnt_c, shared_cnt.at[s])
        plsc.subcore_barrier()
        pltpu.sync_copy(shared_cnt.at[pl.ds(il * NTPI, NTPI)], cnt_cb)
        plsc.subcore_barrier()

    def group_cnt(i):
        return (cnt_cb[0, pl.ds(i * 16, 16)] + cnt_cb[1, pl.ds(i * 16, 16)]
                + cnt_cb[2, pl.ds(i * 16, 16)] + cnt_cb[3, pl.ds(i * 16, 16)])

    def find_bin(nb, kk):
        def fbody(i, carry):
            cum, found, binf, cbel = carry
            cvec = group_cnt(i)
            inc = plsc.cumsum(cvec)
            hit = (inc + cum) >= kk
            first = jnp.min(jnp.where(hit, iota16, jnp.int32(16)))
            newhit = jnp.logical_and(first < 16, found == 0)
            exc_c = jnp.sum(jnp.where(iota16 < first, cvec, jnp.int32(0)))
            binf = jnp.where(newhit, i * 16 + first, binf)
            cbel = jnp.where(newhit, cum + exc_c, cbel)
            found = jnp.where(newhit, jnp.int32(1), found)
            cum = cum + jnp.sum(cvec)
            return cum, found, binf, cbel
        init = (jnp.int32(0), jnp.int32(0), jnp.int32(0), jnp.int32(0))
        out = lax.fori_loop(0, nb // 16, fbody, init)
        return out[2], out[3]

    zero_hist(NBINS[0])

    def piece_body(j, _):
        off = base + j * PIECE
        pltpu.sync_copy(pred_hbm.at[pl.ds(off, PIECE)], pv)
        pltpu.sync_copy(targ_hbm.at[pl.ds(off, PIECE)], tv)
        pltpu.sync_copy(mask_hbm.at[pl.ds(off, PIECE)], mv)

        def vbody(i, _):
            p = pv[pl.ds(i * 16, 16)]
            t = tv[pl.ds(i * 16, 16)]
            m = mv[pl.ds(i * 16, 16)]
            d = jnp.abs(p - t)
            db = lax.bitcast_convert_type(d, jnp.int32)
            valid = m > 0
            eb = jnp.where(valid, db, jnp.int32(SENT))
            errb[pl.ds((j * VECS_PER_PIECE + i) * 16, 16)] = eb
            hist_count(eb, valid, 0)
            return 0
        lax.fori_loop(0, VECS_PER_PIECE, vbody, 0, unroll=8)
        return 0

    lax.fori_loop(0, NPIECE, piece_body, 0)
    publish_and_combine(NBINS[0])

    def mbody(i, acc):
        return acc + jnp.sum(group_cnt(i))
    M = lax.fori_loop(0, NBINS[0] // 16, mbody, jnp.int32(0))
    Mf = M.astype(jnp.float32)
    k = (Mf * jnp.float32(0.8)).astype(jnp.int32)

    b0, c0 = find_bin(NBINS[0], k)
    kk = k - c0
    prefix = b0
    T0 = b0 * jnp.int32(1 << 22)

    zero_hist(NBINS[1])

    def l1body(i, carry):
        off, vsum = carry
        eb = errb[pl.ds(i * 16, 16)]
        match = jnp.logical_and(
            eb != jnp.int32(SENT),
            jnp.right_shift(eb, PCHK[1]) == prefix)
        hist_count(eb, match, 1)
        plsc.store_compressed(cand.at[pl.ds(off, 16)], eb, mask=match)
        vals = lax.bitcast_convert_type(eb, jnp.float32)
        vsum = vsum + jnp.where(eb < T0, vals, zeros_f)
        off = off + jnp.max(plsc.all_reduce_population_count(match))
        return off, vsum
    nc, vsum0 = lax.fori_loop(0, CHUNK // 16, l1body,
                              (jnp.int32(0), zeros_f))
    publish_and_combine(NBINS[1])
    b1, c1 = find_bin(NBINS[1], kk)
    kk = kk - c1
    prefix = prefix * jnp.int32(1 << PWID[1]) + b1

    nvec = (nc + jnp.int32(15)) // jnp.int32(16)
    for level in (2, 3):
        zero_hist(NBINS[level])

        def lbody(i, _, _level=level, _prefix=prefix):
            eb = cand[pl.ds(i * 16, 16)]
            inb = (i * 16 + iota16) < nc
            valid = jnp.logical_and(
                inb, jnp.right_shift(eb, PCHK[_level]) == _prefix)
            hist_count(eb, valid, _level)
            return 0
        lax.fori_loop(0, nvec, lbody, 0)
        publish_and_combine(NBINS[level])
        bl, cl = find_bin(NBINS[level], kk)
        kk = kk - cl
        prefix = prefix * jnp.int32(1 << PWID[level]) + bl

    def sbody(i, vsum):
        eb = cand[pl.ds(i * 16, 16)]
        inb = (i * 16 + iota16) < nc
        sel = jnp.logical_and(inb, eb < prefix)
        vals = lax.bitcast_convert_type(eb, jnp.float32)
        return vsum + jnp.where(sel, vals, zeros_f)
    vsum1 = lax.fori_loop(0, nvec, sbody, zeros_f)
    spart = jnp.sum(vsum0) + jnp.sum(vsum1)

    psum_pub[...] = jnp.where(iota16 == 0, spart, zeros_f)
    pltpu.sync_copy(psum_pub, shared_psum.at[s])
    plsc.subcore_barrier()
    pltpu.sync_copy(shared_psum.at[pl.ds(il * NTPI, NTPI)], psum_cb)
    sum_below = jnp.sum(psum_cb[0] + psum_cb[1] + psum_cb[2] + psum_cb[3])

    t = jnp.max(lax.bitcast_convert_type(
        jnp.full((16,), prefix, jnp.int32), jnp.float32))
    image_sum = sum_below + kk.astype(jnp.float32) * t

    @pl.when(tg == 0)
    def _():
        outv[...] = jnp.where(iota16 == 0, image_sum,
                              jnp.where(iota16 == 1, Mf, jnp.float32(0.0)))
        pltpu.sync_copy(outv, out_hbm.at[img])


def kernel(prediction, target, mask):
    pf = prediction.reshape(-1)
    tf = target.reshape(-1)
    mf = mask.reshape(-1)
    out = _tmae_sc(pf, tf, mf)
    image_sum = out[:, 0]
    M = out[:, 1]
    divisor = jnp.sum(M * jnp.float32(0.8))
    return jnp.where(divisor == 0,
                     jnp.float32(0.0),
                     jnp.sum(image_sum) / jnp.maximum(divisor, jnp.float32(1e-12)))
